# baseline (speedup 1.0000x reference)
import functools

import jax
import jax.numpy as jnp
from jax import lax
from jax.experimental import pallas as pl
from jax.experimental.pallas import tpu as pltpu
from jax.experimental.pallas import tpu_sc as plsc

VOCAB = 100000
EMBED = 128
B, T = 16, 2048
ROWS = B * T
NW = 32
ROWS_PER_W = ROWS // NW
CHUNK = 128
NCHUNK = ROWS_PER_W // CHUNK
LANES = 16


def _body(idx_hbm, tok_hbm, pos_hbm, out_hbm, idx_v, rows_v, pos_v, sem):
    wid = lax.axis_index("s") * 2 + lax.axis_index("c")
    base = wid * ROWS_PER_W
    t0 = (wid % 2) * ROWS_PER_W

    pltpu.sync_copy(idx_hbm.at[wid], idx_v)

    for j in range(NCHUNK):
        pltpu.async_copy(tok_hbm.at[idx_v.at[j]], rows_v, sem).wait()
        pltpu.sync_copy(pos_hbm.at[pl.ds(t0 + j * CHUNK, CHUNK)], pos_v)

        def add_row(r):
            for c in range(EMBED // LANES):
                sl = pl.ds(c * LANES, LANES)
                rows_v[r, sl] = rows_v[r, sl] + pos_v[r, sl]

        lax.fori_loop(0, CHUNK, lambda r, _: (add_row(r), 0)[1], 0)

        pltpu.sync_copy(rows_v, out_hbm.at[pl.ds(base + j * CHUNK, CHUNK)])


def kernel(idx, tok_table, pos_table):
    idx3 = idx.reshape(NW, NCHUNK, CHUNK).astype(jnp.int32)
    mesh = plsc.VectorSubcoreMesh(core_axis_name="c", subcore_axis_name="s")
    out = pl.kernel(
        _body,
        mesh=mesh,
        out_type=jax.ShapeDtypeStruct((ROWS, EMBED), jnp.float32),
        scratch_types=[
            pltpu.VMEM((NCHUNK, CHUNK), jnp.int32),
            pltpu.VMEM((CHUNK, EMBED), jnp.float32),
            pltpu.VMEM((CHUNK, EMBED), jnp.float32),
            pltpu.SemaphoreType.DMA,
        ],
    )(idx3, tok_table, pos_table)
    return out.reshape(B, T, EMBED)


# speedup vs baseline: 1.6437x; 1.6437x over previous
import jax
import jax.numpy as jnp
from jax import lax
from jax.experimental import pallas as pl
from jax.experimental.pallas import tpu as pltpu
from jax.experimental.pallas import tpu_sc as plsc

VOCAB = 100000
EMBED = 128
B, T = 16, 2048
ROWS = B * T
NW = 32
TW = T // NW
NB = 4
LOOKAHEAD = 2


def _body(idx_hbm, tok_hbm, pos_hbm, out_hbm,
          idx_v, pos_v,
          r0, r1, r2, r3, g0, g1, g2, g3, s0, s1, s2, s3):
    rows = [r0, r1, r2, r3]
    gsem = [g0, g1, g2, g3]
    ssem = [s0, s1, s2, s3]
    wid = lax.axis_index("s") * 2 + lax.axis_index("c")
    t0 = wid * TW

    pltpu.sync_copy(idx_hbm.at[wid], idx_v)
    pltpu.sync_copy(pos_hbm.at[pl.ds(t0, TW)], pos_v)

    g = {}
    s = {}

    def start_gather(b):
        buf = b % NB
        g[b] = pltpu.async_copy(tok_hbm.at[idx_v.at[b]], rows[buf], gsem[buf])

    for b in range(LOOKAHEAD):
        start_gather(b)

    for b in range(B):
        buf = b % NB
        nb = b + LOOKAHEAD
        if nb < B:
            pb = nb - NB
            if pb >= 0:
                s[pb].wait()
            start_gather(nb)
        g[b].wait()

        def row_body(r, _, buf=buf):
            for c in range(EMBED // 16):
                sl = pl.ds(c * 16, 16)
                plsc.addupdate(rows[buf].at[r, sl], pos_v[r, sl])
            return 0

        lax.fori_loop(0, TW, row_body, 0, unroll=2)
        s[b] = pltpu.async_copy(
            rows[buf], out_hbm.at[pl.ds(b * T + t0, TW)], ssem[buf])

    for b in range(B - NB, B):
        s[b].wait()


def kernel(idx, tok_table, pos_table):
    idx_r = idx.astype(jnp.int32).reshape(B, NW, TW).transpose(1, 0, 2)
    mesh = plsc.VectorSubcoreMesh(core_axis_name="c", subcore_axis_name="s")
    out = pl.kernel(
        _body,
        mesh=mesh,
        out_type=jax.ShapeDtypeStruct((ROWS, EMBED), jnp.float32),
        scratch_types=[
            pltpu.VMEM((B, TW), jnp.int32),
            pltpu.VMEM((TW, EMBED), jnp.float32),
        ] + [pltpu.VMEM((TW, EMBED), jnp.float32)] * NB
          + [pltpu.SemaphoreType.DMA] * (2 * NB),
    )(idx_r, tok_table, pos_table)
    return out.reshape(B, T, EMBED)


# speedup vs baseline: 1.7058x; 1.0378x over previous
import jax
import jax.numpy as jnp
from jax import lax
from jax.experimental import pallas as pl
from jax.experimental.pallas import tpu as pltpu
from jax.experimental.pallas import tpu_sc as plsc

VOCAB = 100000
EMBED = 128
B, T = 16, 2048
ROWS = B * T
NW = 32
TW = T // NW
CB = 2
CROWS = CB * TW
NCH = B // CB
NB = 4
LOOKAHEAD = 2


def _body(idx_hbm, tok_hbm, pos_hbm, out_hbm,
          idx_v, pos_v,
          r0, r1, r2, r3, g0, g1, g2, g3, s0, s1, s2, s3, psem):
    rows = [r0, r1, r2, r3]
    gsem = [g0, g1, g2, g3]
    ssem = [s0, s1, s2, s3]
    wid = lax.axis_index("s") * 2 + lax.axis_index("c")
    t0 = wid * TW

    c_pos = pltpu.async_copy(pos_hbm.at[pl.ds(t0, TW)], pos_v, psem)
    pltpu.sync_copy(idx_hbm.at[wid], idx_v)
    c_pos.wait()

    g = {}
    s = {}

    def start_gather(j):
        buf = j % NB
        g[j] = pltpu.async_copy(tok_hbm.at[idx_v.at[j]], rows[buf], gsem[buf])

    for j in range(LOOKAHEAD):
        start_gather(j)

    for j in range(NCH):
        buf = j % NB
        nj = j + LOOKAHEAD
        if nj < NCH:
            pj = nj - NB
            if pj >= 0:
                s[pj][0].wait()
                s[pj][1].wait()
            start_gather(nj)
        g[j].wait()

        def row_body(r, _, buf=buf):
            for c in range(EMBED // 16):
                sl = pl.ds(c * 16, 16)
                v = pos_v[r, sl]
                for k in range(CB):
                    plsc.addupdate(rows[buf].at[k * TW + r, sl], v)
            return 0

        lax.fori_loop(0, TW, row_body, 0, unroll=2)

        s[j] = tuple(
            pltpu.async_copy(
                rows[buf].at[pl.ds(k * TW, TW)],
                out_hbm.at[pl.ds((j * CB + k) * T + t0, TW)],
                ssem[buf])
            for k in range(CB))

    for j in range(NCH - NB, NCH):
        s[j][0].wait()
        s[j][1].wait()


def kernel(idx, tok_table, pos_table):
    idx_r = (idx.astype(jnp.int32)
             .reshape(NCH, CB, NW, TW)
             .transpose(2, 0, 1, 3)
             .reshape(NW, NCH, CROWS))
    mesh = plsc.VectorSubcoreMesh(core_axis_name="c", subcore_axis_name="s")
    out = pl.kernel(
        _body,
        mesh=mesh,
        out_type=jax.ShapeDtypeStruct((ROWS, EMBED), jnp.float32),
        scratch_types=[
            pltpu.VMEM((NCH, CROWS), jnp.int32),
            pltpu.VMEM((TW, EMBED), jnp.float32),
        ] + [pltpu.VMEM((CROWS, EMBED), jnp.float32)] * NB
          + [pltpu.SemaphoreType.DMA] * (2 * NB + 1),
    )(idx_r, tok_table, pos_table)
    return out.reshape(B, T, EMBED)
